# baseline (speedup 1.0000x reference)
import functools

import jax
import jax.numpy as jnp
from jax import lax
from jax.experimental import pallas as pl
from jax.experimental.pallas import tpu as pltpu
from jax.experimental.pallas import tpu_sc as plsc

_NC = 2
_NS = 16
_NW = _NC * _NS
_K = 128


def _project_body(x_ref, wl_ref, wr_ref, xl_ref, xr_ref):
    xb = x_ref[...]
    xl_ref[...] = jnp.dot(xb, wl_ref[...], preferred_element_type=jnp.float32)
    xr_ref[...] = jnp.dot(xb, wr_ref[...], preferred_element_type=jnp.float32)


def _project(x, wl_p, wr_p):
    n, f = x.shape
    br = 1000
    return pl.pallas_call(
        _project_body,
        grid=(n // br,),
        in_specs=[
            pl.BlockSpec((br, f), lambda i: (i, 0)),
            pl.BlockSpec((f, 16), lambda i: (0, 0)),
            pl.BlockSpec((f, 16), lambda i: (0, 0)),
        ],
        out_specs=[
            pl.BlockSpec((br, 16), lambda i: (i, 0)),
            pl.BlockSpec((br, 16), lambda i: (i, 0)),
        ],
        out_shape=[
            jax.ShapeDtypeStruct((n, 16), jnp.float32),
            jax.ShapeDtypeStruct((n, 16), jnp.float32),
        ],
    )(x, wl_p, wr_p)


def _build_edge_kernel(n, nchunks):
    nuni = nchunks // _NW
    extra = nchunks - nuni * _NW
    cw = 640
    cw_last = n - (_NS - 1) * cw
    mesh = plsc.VectorSubcoreMesh(core_axis_name="c", subcore_axis_name="s",
                                  num_cores=_NC, num_subcores=_NS)

    @functools.partial(
        pl.kernel,
        mesh=mesh,
        out_type=jax.ShapeDtypeStruct((_NC, 11, n), jnp.float32),
        scratch_types=[
            pltpu.VMEM((11, n), jnp.float32),
            pltpu.VMEM((_K,), jnp.int32),
            pltpu.VMEM((_K,), jnp.int32),
            pltpu.VMEM((_K,), jnp.int32),
            pltpu.VMEM((_K,), jnp.int32),
            pltpu.VMEM((_K, 16), jnp.float32),
            pltpu.VMEM((_K, 16), jnp.float32),
            pltpu.VMEM((_K, 16), jnp.float32),
            pltpu.VMEM((_K, 16), jnp.float32),
            pltpu.VMEM((10, 16), jnp.float32),
            pltpu.VMEM((11,), jnp.int32),
            pltpu.VMEM_SHARED((11, n), jnp.float32),
            pltpu.SemaphoreType.DMA,
            pltpu.SemaphoreType.DMA,
            pltpu.SemaphoreType.DMA,
            pltpu.SemaphoreType.DMA,
            pltpu.SemaphoreType.DMA,
            pltpu.SemaphoreType.DMA,
            pltpu.SemaphoreType.DMA,
            pltpu.SemaphoreType.DMA,
            pltpu.SemaphoreType.DMA,
        ],
        compiler_params=pltpu.CompilerParams(needs_layout_passes=False,
                                             use_tc_tiling_on_sc=False),
    )
    def edge_kernel(xl_hbm, xr_hbm, ei_hbm, att_hbm, i11_hbm, out_hbm,
                    acc, si0, si1, di0, di1, xl0, xl1, xr0, xr1,
                    att_v, i11_v, acc_sh,
                    pi0, pi1, pj0, pj1, sa0, sa1, sb0, sb1, sm):
        cid = lax.axis_index("c")
        sid = lax.axis_index("s")
        si_b = (si0, si1)
        di_b = (di0, di1)
        xl_b = (xl0, xl1)
        xr_b = (xr0, xr1)
        pi = (pi0, pi1)
        pj = (pj0, pj1)
        sa = (sa0, sa1)
        sb = (sb0, sb1)
        dummy_i = ei_hbm.at[0, 0]
        dummy_g = xl_hbm.at[pl.ds(0, _K)]

        rank = sid * _NC + cid
        has_extra = rank < extra
        start = rank * nuni + jnp.minimum(rank, extra)

        pltpu.sync_copy(att_hbm, att_v)
        pltpu.sync_copy(i11_hbm, i11_v)

        for b in range(2):
            pltpu.async_copy(ei_hbm.at[0, start + b], si_b[b], pi[b])
            pltpu.async_copy(ei_hbm.at[1, start + b], di_b[b], pj[b])

        zvec = jnp.zeros((16,), jnp.float32)

        def zrow(i, carry):
            for j in range(11):
                acc[j, pl.ds(i * 16, 16)] = zvec
            return carry

        lax.fori_loop(0, n // 16, zrow, 0)

        cbase = sid * cw

        @pl.when(sid < _NS - 1)
        def _():
            pltpu.sync_copy(acc.at[:, pl.ds(0, cw)],
                            acc_sh.at[:, pl.ds(cbase, cw)])

        @pl.when(sid == _NS - 1)
        def _():
            pltpu.sync_copy(acc.at[:, pl.ds(0, cw_last)],
                            acc_sh.at[:, pl.ds((_NS - 1) * cw, cw_last)])

        att_s = [att_v[j, :] for j in range(10)]

        pltpu.make_async_copy(dummy_i, si0, pi0).wait()
        pltpu.make_async_copy(dummy_i, di0, pj0).wait()
        pltpu.async_copy(xl_hbm.at[si0], xl0, sa0)
        pltpu.async_copy(xr_hbm.at[di0], xr0, sb0)

        def compute(xl_rows, xr_rows, d_idx):
            for g in range(_K // 16):
                e_idx = lax.iota(jnp.int32, 16) + (g * 16)
                dst_v = d_idx[pl.ds(g * 16, 16)]
                logit = jnp.zeros((16,), jnp.float32)
                avs = []
                for j in range(10):
                    jv = jnp.full((16,), j, jnp.int32)
                    av = plsc.load_gather(xl_rows, [e_idx, jv])
                    bv = plsc.load_gather(xr_rows, [e_idx, jv])
                    s = av + bv
                    f = jnp.maximum(s, 0.2 * s)
                    logit = logit + f * att_s[j]
                    avs.append(av)
                w = jnp.exp(logit)
                for j in range(10):
                    plsc.addupdate_scatter(
                        acc, [jnp.full((16,), j, jnp.int32), dst_v],
                        w * avs[j])
                plsc.addupdate_scatter(
                    acc, [jnp.full((16,), 10, jnp.int32), dst_v], w)

        def pair(i, carry):
            r0 = i * 2
            for b in range(2):
                r = r0 + b
                b1 = 1 - b

                @pl.when(r < nuni)
                def _(b=b, b1=b1, r=r):
                    pltpu.make_async_copy(dummy_g, xl_b[b], sa[b]).wait()
                    pltpu.make_async_copy(dummy_g, xr_b[b], sb[b]).wait()

                    @pl.when(r + 1 < nuni)
                    def _():
                        pltpu.make_async_copy(dummy_i, si_b[b1], pi[b1]).wait()
                        pltpu.make_async_copy(dummy_i, di_b[b1], pj[b1]).wait()
                        pltpu.async_copy(xl_hbm.at[si_b[b1]], xl_b[b1], sa[b1])
                        pltpu.async_copy(xr_hbm.at[di_b[b1]], xr_b[b1], sb[b1])

                    compute(xl_b[b], xr_b[b], di_b[b])

                    @pl.when(r + 2 < nuni)
                    def _():
                        pltpu.async_copy(ei_hbm.at[0, start + r + 2],
                                         si_b[b], pi[b])
                        pltpu.async_copy(ei_hbm.at[1, start + r + 2],
                                         di_b[b], pj[b])
            return carry

        lax.fori_loop(0, (nuni + 1) // 2, pair, 0)

        @pl.when(has_extra)
        def _():
            pltpu.async_copy(ei_hbm.at[0, start + nuni], si0, pi0).wait()
            pltpu.async_copy(ei_hbm.at[1, start + nuni], di0, pj0).wait()
            pltpu.async_copy(xl_hbm.at[si0], xl0, sa0).wait()
            pltpu.async_copy(xr_hbm.at[di0], xr0, sb0).wait()
            compute(xl0, xr0, di0)

        plsc.subcore_barrier()

        pltpu.async_copy(acc, acc_sh.at[i11_v], sm, add=True).wait()
        plsc.subcore_barrier()

        @pl.when(sid < _NS - 1)
        def _():
            pltpu.sync_copy(acc_sh.at[:, pl.ds(cbase, cw)],
                            out_hbm.at[cid, :, pl.ds(cbase, cw)])

        @pl.when(sid == _NS - 1)
        def _():
            pltpu.sync_copy(acc_sh.at[:, pl.ds((_NS - 1) * cw, cw_last)],
                            out_hbm.at[cid, :, pl.ds((_NS - 1) * cw, cw_last)])

    return edge_kernel


def _final_body(acc_ref, batch_ref, bias_ref, wct_ref, bc_ref,
                out_ref, pooled_ref):
    acc = acc_ref[0] + acc_ref[1]
    s = acc[10:11, :]
    h = jnp.maximum(acc[0:10, :] / jnp.clip(s, 1e-16) + bias_ref[...], 0.0)
    n = h.shape[1]
    oh = (batch_ref[...] == lax.broadcasted_iota(jnp.int32, (64, n), 0))
    oh = oh.astype(jnp.float32)
    pooled_sum = jax.lax.dot_general(
        h, oh, (((1,), (1,)), ((), ())), preferred_element_type=jnp.float32)
    counts = jnp.sum(oh, axis=1)[None, :]
    pooled = pooled_sum / jnp.clip(counts, 1.0)
    logits = jnp.dot(wct_ref[...], pooled,
                     preferred_element_type=jnp.float32) + bc_ref[...]
    m = jnp.max(logits, axis=0, keepdims=True)
    e = jnp.exp(logits - m)
    out_ref[...] = e / jnp.sum(e, axis=0, keepdims=True)
    pooled_ref[...] = pooled


def _final(acc2, batch_r, bias_c, wc_t, bc_c):
    return pl.pallas_call(
        _final_body,
        out_shape=[
            jax.ShapeDtypeStruct((10, 64), jnp.float32),
            jax.ShapeDtypeStruct((10, 64), jnp.float32),
        ],
    )(acc2, batch_r, bias_c, wc_t, bc_c)


def kernel(x, edge_index, batch, W_l, W_r, att, bias, W_c, b_c):
    n, f = x.shape
    h = W_l.shape[1]
    e = edge_index.shape[1]
    c = W_c.shape[1]
    nchunks = e // _K

    wl_p = jnp.pad(W_l, ((0, 0), (0, 16 - h)))
    wr_p = jnp.pad(W_r, ((0, 0), (0, 16 - h)))
    xlp, xrp = _project(x, wl_p, wr_p)

    ei3 = edge_index.reshape(2, nchunks, _K)
    att_p = jnp.broadcast_to(att[:, None], (h, 16))
    i11 = jnp.arange(11, dtype=jnp.int32)

    acc2 = _build_edge_kernel(n, nchunks)(xlp, xrp, ei3, att_p, i11)

    batch_r = batch[None, :]
    bias_c = bias[:, None]
    wc_t = W_c.T
    bc_c = b_c[:, None]
    out_t, pooled_t = _final(acc2, batch_r, bias_c, wc_t, bc_c)
    return out_t.T, pooled_t.T


# speedup vs baseline: 1.2284x; 1.2284x over previous
import functools

import jax
import jax.numpy as jnp
from jax import lax
from jax.experimental import pallas as pl
from jax.experimental.pallas import tpu as pltpu
from jax.experimental.pallas import tpu_sc as plsc

_NC = 2
_NS = 16
_NW = _NC * _NS
_K = 128
_NB = 2


def _project_body(x_ref, wl_ref, wr_ref, xl_ref, xr_ref):
    xb = x_ref[...]
    l = jnp.dot(xb, wl_ref[...], preferred_element_type=jnp.float32)
    r = jnp.dot(xb, wr_ref[...], preferred_element_type=jnp.float32)
    col = lax.broadcasted_iota(jnp.int32, l.shape, 1)
    xl_ref[...] = jnp.where(col == 10, 1.0, l)
    xr_ref[...] = r


def _project(x, wl_p, wr_p):
    n, f = x.shape
    br = 1000
    return pl.pallas_call(
        _project_body,
        grid=(n // br,),
        in_specs=[
            pl.BlockSpec((br, f), lambda i: (i, 0)),
            pl.BlockSpec((f, 16), lambda i: (0, 0)),
            pl.BlockSpec((f, 16), lambda i: (0, 0)),
        ],
        out_specs=[
            pl.BlockSpec((br, 16), lambda i: (i, 0)),
            pl.BlockSpec((br, 16), lambda i: (i, 0)),
        ],
        out_shape=[
            jax.ShapeDtypeStruct((n, 16), jnp.float32),
            jax.ShapeDtypeStruct((n, 16), jnp.float32),
        ],
    )(x, wl_p, wr_p)


def _build_edge_kernel(n, nchunks):
    nuni = nchunks // _NW
    extra = nchunks - nuni * _NW
    rows_per_tile = n // _NS
    mesh = plsc.VectorSubcoreMesh(core_axis_name="c", subcore_axis_name="s",
                                  num_cores=_NC, num_subcores=_NS)

    @functools.partial(
        pl.kernel,
        mesh=mesh,
        out_type=jax.ShapeDtypeStruct((_NC, n, 16), jnp.float32),
        scratch_types=[
            pltpu.VMEM((nuni, _K), jnp.int32),
            pltpu.VMEM((nuni, _K), jnp.int32),
            pltpu.VMEM((_K,), jnp.int32),
            pltpu.VMEM((_K,), jnp.int32),
        ] + [pltpu.VMEM((_K, 16), jnp.float32)] * (3 * _NB) + [
            pltpu.VMEM((10, 16), jnp.float32),
            pltpu.VMEM((rows_per_tile, 16), jnp.float32),
            pltpu.VMEM_SHARED((n, 16), jnp.float32),
        ] + [pltpu.SemaphoreType.DMA] * (3 * _NB),
        compiler_params=pltpu.CompilerParams(needs_layout_passes=False,
                                             use_tc_tiling_on_sc=False),
    )
    def edge_kernel(xl_hbm, xr_hbm, ei_hbm, att_hbm, out_hbm,
                    src_buf, dst_buf, src_t, dst_t, *rest):
        xl_b = rest[0:_NB]
        xr_b = rest[_NB:2 * _NB]
        or_b = rest[2 * _NB:3 * _NB]
        att_v = rest[3 * _NB]
        zbuf = rest[3 * _NB + 1]
        acc_sh = rest[3 * _NB + 2]
        sa = rest[3 * _NB + 3:4 * _NB + 3]
        sb = rest[4 * _NB + 3:5 * _NB + 3]
        ss = rest[5 * _NB + 3:6 * _NB + 3]
        cid = lax.axis_index("c")
        sid = lax.axis_index("s")
        wid = cid * _NS + sid
        dummy = xl_hbm.at[pl.ds(0, _K)]

        rank = sid * _NC + cid
        has_extra = rank < extra
        start = rank * nuni + jnp.minimum(rank, extra)

        pltpu.sync_copy(att_hbm, att_v)
        pltpu.sync_copy(ei_hbm.at[0, pl.ds(start, nuni)], src_buf)
        pltpu.sync_copy(ei_hbm.at[1, pl.ds(start, nuni)], dst_buf)

        @pl.when(has_extra)
        def _():
            pltpu.sync_copy(ei_hbm.at[0, start + nuni], src_t)
            pltpu.sync_copy(ei_hbm.at[1, start + nuni], dst_t)

        zvec = jnp.zeros((16,), jnp.float32)

        def zrow(i, carry):
            zbuf[i, :] = zvec
            return carry

        lax.fori_loop(0, rows_per_tile, zrow, 0)

        def zrow2(i, carry):
            for b in range(_NB):
                or_b[b][i, :] = zvec
            return carry

        lax.fori_loop(0, _K, zrow2, 0)

        pltpu.sync_copy(zbuf, acc_sh.at[pl.ds(sid * rows_per_tile, rows_per_tile)])
        plsc.subcore_barrier()

        att_s = [att_v[j, :] for j in range(10)]

        def compute(xl_rows, xr_rows, out_rows):
            for g in range(_K // 16):
                e_idx = lax.iota(jnp.int32, 16) + (g * 16)
                logit = jnp.zeros((16,), jnp.float32)
                avs = []
                for j in range(10):
                    jv = jnp.full((16,), j, jnp.int32)
                    av = plsc.load_gather(xl_rows, [e_idx, jv])
                    bv = plsc.load_gather(xr_rows, [e_idx, jv])
                    s = av + bv
                    f = jnp.maximum(s, 0.2 * s)
                    logit = logit + f * att_s[j]
                    avs.append(av)
                w = jnp.exp(logit)
                for j in range(10):
                    plsc.store_scatter(
                        out_rows, [e_idx, jnp.full((16,), j, jnp.int32)],
                        w * avs[j])
                plsc.store_scatter(
                    out_rows, [e_idx, jnp.full((16,), 10, jnp.int32)], w)

        for b in range(min(_NB, nuni)):
            pltpu.async_copy(xl_hbm.at[src_buf.at[b]], xl_b[b], sa[b])
            pltpu.async_copy(xr_hbm.at[dst_buf.at[b]], xr_b[b], sb[b])

        def quad(i, carry):
            r0 = i * _NB
            for b in range(_NB):
                r = r0 + b

                @pl.when(r < nuni)
                def _(b=b, r=r):
                    pltpu.make_async_copy(dummy, xl_b[b], sa[b]).wait()
                    pltpu.make_async_copy(dummy, xr_b[b], sb[b]).wait()

                    @pl.when(r >= _NB)
                    def _():
                        pltpu.make_async_copy(dummy, or_b[b], ss[b]).wait()

                    compute(xl_b[b], xr_b[b], or_b[b])

                    pltpu.async_copy(or_b[b], acc_sh.at[dst_buf.at[r]],
                                     ss[b], add=True)

                    @pl.when(r + _NB < nuni)
                    def _():
                        pltpu.async_copy(xl_hbm.at[src_buf.at[r + _NB]],
                                         xl_b[b], sa[b])
                        pltpu.async_copy(xr_hbm.at[dst_buf.at[r + _NB]],
                                         xr_b[b], sb[b])
            return carry

        lax.fori_loop(0, -(-nuni // _NB), quad, 0)

        for b in range(min(_NB, nuni)):
            pltpu.make_async_copy(dummy, or_b[b], ss[b]).wait()

        @pl.when(has_extra)
        def _():
            pltpu.async_copy(xl_hbm.at[src_t], xl_b[0], sa[0]).wait()
            pltpu.async_copy(xr_hbm.at[dst_t], xr_b[0], sb[0]).wait()
            compute(xl_b[0], xr_b[0], or_b[0])
            pltpu.async_copy(or_b[0], acc_sh.at[dst_t], ss[0], add=True).wait()

        plsc.subcore_barrier()
        pltpu.sync_copy(
            acc_sh.at[pl.ds(sid * rows_per_tile, rows_per_tile)],
            out_hbm.at[cid, pl.ds(sid * rows_per_tile, rows_per_tile)])

    return edge_kernel


def _final_body(acc_ref, batch_ref, bias_ref, wc_ref, bc_ref,
                out_ref, pooled_ref):
    acc = acc_ref[0] + acc_ref[1]
    s = acc[:, 10:11]
    h = jnp.maximum(acc / jnp.clip(s, 1e-16) + bias_ref[...], 0.0)
    col = lax.broadcasted_iota(jnp.int32, h.shape, 1)
    h = jnp.where(col < 10, h, 0.0)
    n = h.shape[0]
    oh = (batch_ref[...] == lax.broadcasted_iota(jnp.int32, (64, n), 0))
    oh = oh.astype(jnp.float32)
    pooled_sum = jax.lax.dot_general(
        oh, h, (((1,), (0,)), ((), ())), preferred_element_type=jnp.float32)
    counts = jnp.sum(oh, axis=1, keepdims=True)
    pooled = pooled_sum / jnp.clip(counts, 1.0)
    logits = jnp.dot(pooled, wc_ref[...],
                     preferred_element_type=jnp.float32) + bc_ref[...]
    m = jnp.max(logits, axis=1, keepdims=True)
    e = jnp.exp(logits - m)
    out_ref[...] = e / jnp.sum(e, axis=1, keepdims=True)
    pooled_ref[...] = pooled


def _final(acc2, batch_r, bias_p, wc_p, bc_p):
    return pl.pallas_call(
        _final_body,
        out_shape=[
            jax.ShapeDtypeStruct((64, 16), jnp.float32),
            jax.ShapeDtypeStruct((64, 16), jnp.float32),
        ],
    )(acc2, batch_r, bias_p, wc_p, bc_p)


def kernel(x, edge_index, batch, W_l, W_r, att, bias, W_c, b_c):
    n, f = x.shape
    h = W_l.shape[1]
    e = edge_index.shape[1]
    c = W_c.shape[1]
    nchunks = e // _K

    wl_p = jnp.pad(W_l, ((0, 0), (0, 16 - h)))
    wr_p = jnp.pad(W_r, ((0, 0), (0, 16 - h)))
    xlp, xrp = _project(x, wl_p, wr_p)

    ei3 = edge_index.reshape(2, nchunks, _K)
    att_p = jnp.broadcast_to(att[:, None], (h, 16))

    acc2 = _build_edge_kernel(n, nchunks)(xlp, xrp, ei3, att_p)

    batch_r = batch[None, :]
    bias_p = jnp.pad(bias, (0, 16 - h))[None, :]
    wc_p = jnp.pad(W_c, ((0, 16 - h), (0, 16 - c)))
    bc_p = jnp.pad(b_c, (0, 16 - c), constant_values=-1e30)[None, :]
    out_p, pooled_p = _final(acc2, batch_r, bias_p, wc_p, bc_p)
    return out_p[:, :c], pooled_p[:, :h]


# speedup vs baseline: 1.2630x; 1.0282x over previous
import functools

import jax
import jax.numpy as jnp
from jax import lax
from jax.experimental import pallas as pl
from jax.experimental.pallas import tpu as pltpu
from jax.experimental.pallas import tpu_sc as plsc

_NC = 2
_NS = 16
_NW = _NC * _NS
_K = 128
_NB = 2


def _project_body(x_ref, wl_ref, wr_ref, xl_ref, xr_ref):
    xb = x_ref[...]
    l = jnp.dot(xb, wl_ref[...], preferred_element_type=jnp.float32)
    r = jnp.dot(xb, wr_ref[...], preferred_element_type=jnp.float32)
    col = lax.broadcasted_iota(jnp.int32, l.shape, 1)
    xl_ref[...] = jnp.where(col == 10, 1.0, l)
    xr_ref[...] = r


def _project(x, wl_p, wr_p):
    n, f = x.shape
    br = 1000
    return pl.pallas_call(
        _project_body,
        grid=(n // br,),
        in_specs=[
            pl.BlockSpec((br, f), lambda i: (i, 0)),
            pl.BlockSpec((f, 16), lambda i: (0, 0)),
            pl.BlockSpec((f, 16), lambda i: (0, 0)),
        ],
        out_specs=[
            pl.BlockSpec((br, 16), lambda i: (i, 0)),
            pl.BlockSpec((br, 16), lambda i: (i, 0)),
        ],
        out_shape=[
            jax.ShapeDtypeStruct((n, 16), jnp.float32),
            jax.ShapeDtypeStruct((n, 16), jnp.float32),
        ],
    )(x, wl_p, wr_p)


def _build_edge_kernel(n, nchunks):
    nuni = nchunks // _NW
    extra = nchunks - nuni * _NW
    rows_per_tile = n // _NS
    mesh = plsc.VectorSubcoreMesh(core_axis_name="c", subcore_axis_name="s",
                                  num_cores=_NC, num_subcores=_NS)

    @functools.partial(
        pl.kernel,
        mesh=mesh,
        out_type=jax.ShapeDtypeStruct((_NC, n, 16), jnp.float32),
        scratch_types=[
            pltpu.VMEM((nuni, _K), jnp.int32),
            pltpu.VMEM((nuni, _K), jnp.int32),
            pltpu.VMEM((_K,), jnp.int32),
            pltpu.VMEM((_K,), jnp.int32),
        ] + [pltpu.VMEM((_K, 16), jnp.float32)] * (3 * _NB) + [
            pltpu.VMEM((10, 16), jnp.float32),
            pltpu.VMEM((rows_per_tile, 16), jnp.float32),
            pltpu.VMEM_SHARED((n, 16), jnp.float32),
            pltpu.VMEM_SHARED((n, 16), jnp.float32),
            pltpu.VMEM_SHARED((n, 16), jnp.float32),
        ] + [pltpu.SemaphoreType.DMA] * (3 * _NB),
        compiler_params=pltpu.CompilerParams(needs_layout_passes=False,
                                             use_tc_tiling_on_sc=False),
    )
    def edge_kernel(xl_hbm, xr_hbm, ei_hbm, att_hbm, out_hbm,
                    src_buf, dst_buf, src_t, dst_t, *rest):
        xl_b = rest[0:_NB]
        xr_b = rest[_NB:2 * _NB]
        or_b = rest[2 * _NB:3 * _NB]
        att_v = rest[3 * _NB]
        zbuf = rest[3 * _NB + 1]
        acc_sh = rest[3 * _NB + 2]
        xl_sh = rest[3 * _NB + 3]
        xr_sh = rest[3 * _NB + 4]
        sa = rest[3 * _NB + 5:4 * _NB + 5]
        sb = rest[4 * _NB + 5:5 * _NB + 5]
        ss = rest[5 * _NB + 5:6 * _NB + 5]
        cid = lax.axis_index("c")
        sid = lax.axis_index("s")
        wid = cid * _NS + sid
        dummy = xl_hbm.at[pl.ds(0, _K)]

        rank = sid * _NC + cid
        has_extra = rank < extra
        start = rank * nuni + jnp.minimum(rank, extra)

        pltpu.sync_copy(att_hbm, att_v)
        trow = sid * rows_per_tile
        pltpu.sync_copy(xl_hbm.at[pl.ds(trow, rows_per_tile)],
                        xl_sh.at[pl.ds(trow, rows_per_tile)])
        pltpu.sync_copy(xr_hbm.at[pl.ds(trow, rows_per_tile)],
                        xr_sh.at[pl.ds(trow, rows_per_tile)])
        pltpu.sync_copy(ei_hbm.at[0, pl.ds(start, nuni)], src_buf)
        pltpu.sync_copy(ei_hbm.at[1, pl.ds(start, nuni)], dst_buf)

        @pl.when(has_extra)
        def _():
            pltpu.sync_copy(ei_hbm.at[0, start + nuni], src_t)
            pltpu.sync_copy(ei_hbm.at[1, start + nuni], dst_t)

        zvec = jnp.zeros((16,), jnp.float32)

        def zrow(i, carry):
            zbuf[i, :] = zvec
            return carry

        lax.fori_loop(0, rows_per_tile, zrow, 0)

        def zrow2(i, carry):
            for b in range(_NB):
                or_b[b][i, :] = zvec
            return carry

        lax.fori_loop(0, _K, zrow2, 0)

        pltpu.sync_copy(zbuf, acc_sh.at[pl.ds(sid * rows_per_tile, rows_per_tile)])
        plsc.subcore_barrier()

        att_s = [att_v[j, :] for j in range(10)]

        def compute(xl_rows, xr_rows, out_rows):
            for g in range(_K // 16):
                e_idx = lax.iota(jnp.int32, 16) + (g * 16)
                logit = jnp.zeros((16,), jnp.float32)
                avs = []
                for j in range(10):
                    jv = jnp.full((16,), j, jnp.int32)
                    av = plsc.load_gather(xl_rows, [e_idx, jv])
                    bv = plsc.load_gather(xr_rows, [e_idx, jv])
                    s = av + bv
                    f = jnp.maximum(s, 0.2 * s)
                    logit = logit + f * att_s[j]
                    avs.append(av)
                w = jnp.exp(logit)
                for j in range(10):
                    plsc.store_scatter(
                        out_rows, [e_idx, jnp.full((16,), j, jnp.int32)],
                        w * avs[j])
                plsc.store_scatter(
                    out_rows, [e_idx, jnp.full((16,), 10, jnp.int32)], w)

        for b in range(min(_NB, nuni)):
            pltpu.async_copy(xl_sh.at[src_buf.at[b]], xl_b[b], sa[b])
            pltpu.async_copy(xr_sh.at[dst_buf.at[b]], xr_b[b], sb[b])

        def quad(i, carry):
            r0 = i * _NB
            for b in range(_NB):
                r = r0 + b

                @pl.when(r < nuni)
                def _(b=b, r=r):
                    pltpu.make_async_copy(dummy, xl_b[b], sa[b]).wait()
                    pltpu.make_async_copy(dummy, xr_b[b], sb[b]).wait()

                    @pl.when(r >= _NB)
                    def _():
                        pltpu.make_async_copy(dummy, or_b[b], ss[b]).wait()

                    compute(xl_b[b], xr_b[b], or_b[b])

                    pltpu.async_copy(or_b[b], acc_sh.at[dst_buf.at[r]],
                                     ss[b], add=True)

                    @pl.when(r + _NB < nuni)
                    def _():
                        pltpu.async_copy(xl_sh.at[src_buf.at[r + _NB]],
                                         xl_b[b], sa[b])
                        pltpu.async_copy(xr_sh.at[dst_buf.at[r + _NB]],
                                         xr_b[b], sb[b])
            return carry

        lax.fori_loop(0, -(-nuni // _NB), quad, 0)

        for b in range(min(_NB, nuni)):
            pltpu.make_async_copy(dummy, or_b[b], ss[b]).wait()

        @pl.when(has_extra)
        def _():
            pltpu.async_copy(xl_sh.at[src_t], xl_b[0], sa[0]).wait()
            pltpu.async_copy(xr_sh.at[dst_t], xr_b[0], sb[0]).wait()
            compute(xl_b[0], xr_b[0], or_b[0])
            pltpu.async_copy(or_b[0], acc_sh.at[dst_t], ss[0], add=True).wait()

        plsc.subcore_barrier()
        pltpu.sync_copy(
            acc_sh.at[pl.ds(sid * rows_per_tile, rows_per_tile)],
            out_hbm.at[cid, pl.ds(sid * rows_per_tile, rows_per_tile)])

    return edge_kernel


def _final_body(acc_ref, batch_ref, bias_ref, wc_ref, bc_ref,
                out_ref, pooled_ref):
    acc = acc_ref[0] + acc_ref[1]
    s = acc[:, 10:11]
    h = jnp.maximum(acc / jnp.clip(s, 1e-16) + bias_ref[...], 0.0)
    col = lax.broadcasted_iota(jnp.int32, h.shape, 1)
    h = jnp.where(col < 10, h, 0.0)
    n = h.shape[0]
    oh = (batch_ref[...] == lax.broadcasted_iota(jnp.int32, (64, n), 0))
    oh = oh.astype(jnp.float32)
    pooled_sum = jax.lax.dot_general(
        oh, h, (((1,), (0,)), ((), ())), preferred_element_type=jnp.float32)
    counts = jnp.sum(oh, axis=1, keepdims=True)
    pooled = pooled_sum / jnp.clip(counts, 1.0)
    logits = jnp.dot(pooled, wc_ref[...],
                     preferred_element_type=jnp.float32) + bc_ref[...]
    m = jnp.max(logits, axis=1, keepdims=True)
    e = jnp.exp(logits - m)
    out_ref[...] = e / jnp.sum(e, axis=1, keepdims=True)
    pooled_ref[...] = pooled


def _final(acc2, batch_r, bias_p, wc_p, bc_p):
    return pl.pallas_call(
        _final_body,
        out_shape=[
            jax.ShapeDtypeStruct((64, 16), jnp.float32),
            jax.ShapeDtypeStruct((64, 16), jnp.float32),
        ],
    )(acc2, batch_r, bias_p, wc_p, bc_p)


def kernel(x, edge_index, batch, W_l, W_r, att, bias, W_c, b_c):
    n, f = x.shape
    h = W_l.shape[1]
    e = edge_index.shape[1]
    c = W_c.shape[1]
    nchunks = e // _K

    wl_p = jnp.pad(W_l, ((0, 0), (0, 16 - h)))
    wr_p = jnp.pad(W_r, ((0, 0), (0, 16 - h)))
    xlp, xrp = _project(x, wl_p, wr_p)

    ei3 = edge_index.reshape(2, nchunks, _K)
    att_p = jnp.broadcast_to(att[:, None], (h, 16))

    acc2 = _build_edge_kernel(n, nchunks)(xlp, xrp, ei3, att_p)

    batch_r = batch[None, :]
    bias_p = jnp.pad(bias, (0, 16 - h))[None, :]
    wc_p = jnp.pad(W_c, ((0, 16 - h), (0, 16 - c)))
    bc_p = jnp.pad(b_c, (0, 16 - c), constant_values=-1e30)[None, :]
    out_p, pooled_p = _final(acc2, batch_r, bias_p, wc_p, bc_p)
    return out_p[:, :c], pooled_p[:, :h]


# speedup vs baseline: 1.2785x; 1.0123x over previous
import functools

import jax
import jax.numpy as jnp
from jax import lax
from jax.experimental import pallas as pl
from jax.experimental.pallas import tpu as pltpu
from jax.experimental.pallas import tpu_sc as plsc

_NC = 2
_NS = 16
_NW = _NC * _NS
_K = 128
_NB = 2


def _project_body(x_ref, wl_ref, wr_ref, xl_ref, xr_ref):
    xb = x_ref[...]
    l = jnp.dot(xb, wl_ref[...], preferred_element_type=jnp.float32)
    r = jnp.dot(xb, wr_ref[...], preferred_element_type=jnp.float32)
    col = lax.broadcasted_iota(jnp.int32, l.shape, 1)
    xl_ref[...] = jnp.where(col == 10, 1.0, l)
    xr_ref[...] = r


def _project(x, wl_p, wr_p):
    n, f = x.shape
    br = 1000
    return pl.pallas_call(
        _project_body,
        grid=(n // br,),
        in_specs=[
            pl.BlockSpec((br, f), lambda i: (i, 0)),
            pl.BlockSpec((f, 16), lambda i: (0, 0)),
            pl.BlockSpec((f, 16), lambda i: (0, 0)),
        ],
        out_specs=[
            pl.BlockSpec((br, 16), lambda i: (i, 0)),
            pl.BlockSpec((br, 16), lambda i: (i, 0)),
        ],
        out_shape=[
            jax.ShapeDtypeStruct((n, 16), jnp.float32),
            jax.ShapeDtypeStruct((n, 16), jnp.float32),
        ],
    )(x, wl_p, wr_p)


def _build_edge_kernel(n, nchunks):
    nuni = nchunks // _NW
    extra = nchunks - nuni * _NW
    rows_per_tile = n // _NS
    mesh = plsc.VectorSubcoreMesh(core_axis_name="c", subcore_axis_name="s",
                                  num_cores=_NC, num_subcores=_NS)

    @functools.partial(
        pl.kernel,
        mesh=mesh,
        out_type=jax.ShapeDtypeStruct((_NC, n, 16), jnp.float32),
        scratch_types=[
            pltpu.VMEM((nuni, _K), jnp.int32),
            pltpu.VMEM((nuni, _K), jnp.int32),
            pltpu.VMEM((_K,), jnp.int32),
            pltpu.VMEM((_K,), jnp.int32),
        ] + [pltpu.VMEM((_K, 16), jnp.float32)] * (3 * _NB) + [
            pltpu.VMEM((10, 16), jnp.float32),
            pltpu.VMEM((rows_per_tile, 16), jnp.float32),
            pltpu.VMEM_SHARED((n, 16), jnp.float32),
            pltpu.VMEM_SHARED((n, 16), jnp.float32),
            pltpu.VMEM_SHARED((n, 16), jnp.float32),
        ] + [pltpu.SemaphoreType.DMA] * (3 * _NB),
        compiler_params=pltpu.CompilerParams(needs_layout_passes=False,
                                             use_tc_tiling_on_sc=False),
    )
    def edge_kernel(xl_hbm, xr_hbm, ei_hbm, att_hbm, out_hbm,
                    src_buf, dst_buf, src_t, dst_t, *rest):
        xl_b = rest[0:_NB]
        xr_b = rest[_NB:2 * _NB]
        or_b = rest[2 * _NB:3 * _NB]
        att_v = rest[3 * _NB]
        zbuf = rest[3 * _NB + 1]
        acc_sh = rest[3 * _NB + 2]
        xl_sh = rest[3 * _NB + 3]
        xr_sh = rest[3 * _NB + 4]
        sa = rest[3 * _NB + 5:4 * _NB + 5]
        sb = rest[4 * _NB + 5:5 * _NB + 5]
        ss = rest[5 * _NB + 5:6 * _NB + 5]
        cid = lax.axis_index("c")
        sid = lax.axis_index("s")
        wid = cid * _NS + sid
        dummy = xl_hbm.at[pl.ds(0, _K)]

        rank = sid * _NC + cid
        has_extra = rank < extra
        start = rank * nuni + jnp.minimum(rank, extra)

        pltpu.sync_copy(att_hbm, att_v)
        trow = sid * rows_per_tile
        stg_l = pltpu.async_copy(xl_hbm.at[pl.ds(trow, rows_per_tile)],
                                 xl_sh.at[pl.ds(trow, rows_per_tile)], sa[0])
        stg_r = pltpu.async_copy(xr_hbm.at[pl.ds(trow, rows_per_tile)],
                                 xr_sh.at[pl.ds(trow, rows_per_tile)], sb[0])
        pltpu.sync_copy(ei_hbm.at[0, pl.ds(start, nuni)], src_buf)
        pltpu.sync_copy(ei_hbm.at[1, pl.ds(start, nuni)], dst_buf)

        @pl.when(has_extra)
        def _():
            pltpu.sync_copy(ei_hbm.at[0, start + nuni], src_t)
            pltpu.sync_copy(ei_hbm.at[1, start + nuni], dst_t)

        zvec = jnp.zeros((16,), jnp.float32)

        def zrow(i, carry):
            zbuf[i, :] = zvec
            return carry

        lax.fori_loop(0, rows_per_tile, zrow, 0)

        def zrow2(i, carry):
            for b in range(_NB):
                or_b[b][i, :] = zvec
            return carry

        lax.fori_loop(0, _K, zrow2, 0)

        pltpu.sync_copy(zbuf, acc_sh.at[pl.ds(sid * rows_per_tile, rows_per_tile)])
        stg_l.wait()
        stg_r.wait()
        plsc.subcore_barrier()

        att_s = [att_v[j, :] for j in range(10)]

        def compute(xl_rows, xr_rows, out_rows):
            for g in range(_K // 16):
                e_idx = lax.iota(jnp.int32, 16) + (g * 16)
                logit = jnp.zeros((16,), jnp.float32)
                avs = []
                for j in range(10):
                    jv = jnp.full((16,), j, jnp.int32)
                    av = plsc.load_gather(xl_rows, [e_idx, jv])
                    bv = plsc.load_gather(xr_rows, [e_idx, jv])
                    s = av + bv
                    f = jnp.maximum(s, 0.2 * s)
                    logit = logit + f * att_s[j]
                    avs.append(av)
                w = jnp.exp(logit)
                for j in range(10):
                    plsc.store_scatter(
                        out_rows, [e_idx, jnp.full((16,), j, jnp.int32)],
                        w * avs[j])
                plsc.store_scatter(
                    out_rows, [e_idx, jnp.full((16,), 10, jnp.int32)], w)

        for b in range(min(_NB, nuni)):
            pltpu.async_copy(xl_sh.at[src_buf.at[b]], xl_b[b], sa[b])
            pltpu.async_copy(xr_sh.at[dst_buf.at[b]], xr_b[b], sb[b])

        def quad(i, carry):
            r0 = i * _NB
            for b in range(_NB):
                r = r0 + b

                @pl.when(r < nuni)
                def _(b=b, r=r):
                    pltpu.make_async_copy(dummy, xl_b[b], sa[b]).wait()
                    pltpu.make_async_copy(dummy, xr_b[b], sb[b]).wait()

                    @pl.when(r >= _NB)
                    def _():
                        pltpu.make_async_copy(dummy, or_b[b], ss[b]).wait()

                    compute(xl_b[b], xr_b[b], or_b[b])

                    pltpu.async_copy(or_b[b], acc_sh.at[dst_buf.at[r]],
                                     ss[b], add=True)

                    @pl.when(r + _NB < nuni)
                    def _():
                        pltpu.async_copy(xl_sh.at[src_buf.at[r + _NB]],
                                         xl_b[b], sa[b])
                        pltpu.async_copy(xr_sh.at[dst_buf.at[r + _NB]],
                                         xr_b[b], sb[b])
            return carry

        lax.fori_loop(0, -(-nuni // _NB), quad, 0)

        for b in range(min(_NB, nuni)):
            pltpu.make_async_copy(dummy, or_b[b], ss[b]).wait()

        @pl.when(has_extra)
        def _():
            pltpu.async_copy(xl_sh.at[src_t], xl_b[0], sa[0]).wait()
            pltpu.async_copy(xr_sh.at[dst_t], xr_b[0], sb[0]).wait()
            compute(xl_b[0], xr_b[0], or_b[0])
            pltpu.async_copy(or_b[0], acc_sh.at[dst_t], ss[0], add=True).wait()

        plsc.subcore_barrier()
        pltpu.sync_copy(
            acc_sh.at[pl.ds(sid * rows_per_tile, rows_per_tile)],
            out_hbm.at[cid, pl.ds(sid * rows_per_tile, rows_per_tile)])

    return edge_kernel


def _final_body(acc_ref, batch_ref, bias_ref, wc_ref, bc_ref,
                out_ref, pooled_ref):
    acc = acc_ref[0] + acc_ref[1]
    s = acc[:, 10:11]
    h = jnp.maximum(acc / jnp.clip(s, 1e-16) + bias_ref[...], 0.0)
    col = lax.broadcasted_iota(jnp.int32, h.shape, 1)
    h = jnp.where(col < 10, h, 0.0)
    n = h.shape[0]
    oh = (batch_ref[...] == lax.broadcasted_iota(jnp.int32, (64, n), 0))
    oh = oh.astype(jnp.float32)
    pooled_sum = jax.lax.dot_general(
        oh, h, (((1,), (0,)), ((), ())), preferred_element_type=jnp.float32)
    counts = jnp.sum(oh, axis=1, keepdims=True)
    pooled = pooled_sum / jnp.clip(counts, 1.0)
    logits = jnp.dot(pooled, wc_ref[...],
                     preferred_element_type=jnp.float32) + bc_ref[...]
    m = jnp.max(logits, axis=1, keepdims=True)
    e = jnp.exp(logits - m)
    out_ref[...] = e / jnp.sum(e, axis=1, keepdims=True)
    pooled_ref[...] = pooled


def _final(acc2, batch_r, bias_p, wc_p, bc_p):
    return pl.pallas_call(
        _final_body,
        out_shape=[
            jax.ShapeDtypeStruct((64, 16), jnp.float32),
            jax.ShapeDtypeStruct((64, 16), jnp.float32),
        ],
    )(acc2, batch_r, bias_p, wc_p, bc_p)


def kernel(x, edge_index, batch, W_l, W_r, att, bias, W_c, b_c):
    n, f = x.shape
    h = W_l.shape[1]
    e = edge_index.shape[1]
    c = W_c.shape[1]
    nchunks = e // _K

    wl_p = jnp.pad(W_l, ((0, 0), (0, 16 - h)))
    wr_p = jnp.pad(W_r, ((0, 0), (0, 16 - h)))
    xlp, xrp = _project(x, wl_p, wr_p)

    ei3 = edge_index.reshape(2, nchunks, _K)
    att_p = jnp.broadcast_to(att[:, None], (h, 16))

    acc2 = _build_edge_kernel(n, nchunks)(xlp, xrp, ei3, att_p)

    batch_r = batch[None, :]
    bias_p = jnp.pad(bias, (0, 16 - h))[None, :]
    wc_p = jnp.pad(W_c, ((0, 16 - h), (0, 16 - c)))
    bc_p = jnp.pad(b_c, (0, 16 - c), constant_values=-1e30)[None, :]
    out_p, pooled_p = _final(acc2, batch_r, bias_p, wc_p, bc_p)
    return out_p[:, :c], pooled_p[:, :h]
